# baseline (speedup 1.0000x reference)
import functools

import jax
import jax.numpy as jnp
from jax import lax
from jax.experimental import pallas as pl
from jax.experimental.pallas import tpu as pltpu
from jax.experimental.pallas import tpu_sc as plsc

D = 128
NREL = 32
NC, NS = 2, 16
NW = NC * NS
CHUNK = 128
NBUF = 2
SEG = 16
NSEG = 5
ROWS_PER_TILE = 640
N_PAD = NS * ROWS_PER_TILE
PAD_ROW = 10100


def _sc_partials(entity_emb, head2, tail2, type2, weight):
    cpt = SEG * NSEG
    steps = SEG // NBUF - 1

    mesh = plsc.VectorSubcoreMesh(core_axis_name="c", subcore_axis_name="s")

    @functools.partial(
        pl.kernel,
        out_type=(
            jax.ShapeDtypeStruct((NC, N_PAD, D), jnp.float32),
            jax.ShapeDtypeStruct((NC, N_PAD), jnp.float32),
        ),
        mesh=mesh,
        compiler_params=pltpu.CompilerParams(use_tc_tiling_on_sc=False),
        scratch_types=[
            pltpu.VMEM_SHARED((N_PAD, D), jnp.float32),
            pltpu.VMEM_SHARED((N_PAD,), jnp.float32),
            pltpu.VMEM((NREL, D), jnp.float32),
            pltpu.VMEM((SEG, D), jnp.int32),
            pltpu.VMEM((SEG, D), jnp.int32),
            pltpu.VMEM((SEG, D), jnp.int32),
            pltpu.VMEM((CHUNK, D), jnp.float32),
            pltpu.VMEM((CHUNK, D), jnp.float32),
            pltpu.VMEM((CHUNK,), jnp.float32),
            pltpu.SemaphoreType.DMA,
            pltpu.SemaphoreType.DMA,
            pltpu.SemaphoreType.DMA,
            pltpu.SemaphoreType.DMA,
            pltpu.SemaphoreType.DMA,
        ],
    )
    def agg(emb_h, head_h, tail_h, type_h, w_h, psum_h, pcnt_h,
            acc_s, cnt_s, w_v, tail_v, head_v, type_v,
            rows0, rows1, ones_v,
            g0, g1, s0, s1, csem):
        cid = lax.axis_index("c")
        sid = lax.axis_index("s")
        wid = sid * NC + cid
        rows = [rows0, rows1]
        gsem = [g0, g1]
        ssem = [s0, s1]

        def zrow(i, carry):
            for k in range(D // 16):
                rows0[i, pl.ds(k * 16, 16)] = jnp.zeros((16,), jnp.float32)
            return carry
        lax.fori_loop(0, CHUNK, zrow, 0)
        for i in range(CHUNK // 16):
            ones_v[pl.ds(i * 16, 16)] = jnp.ones((16,), jnp.float32)

        pltpu.sync_copy(w_h, w_v)

        base_row = sid * ROWS_PER_TILE
        for b in range(ROWS_PER_TILE // CHUNK):
            pltpu.sync_copy(rows0, acc_s.at[pl.ds(base_row + b * CHUNK, CHUNK)])
        for b in range(ROWS_PER_TILE // CHUNK):
            pltpu.sync_copy(rows0.at[0],
                            cnt_s.at[pl.ds(base_row + b * CHUNK, CHUNK)])
        plsc.subcore_barrier()

        def compute(buf, j):
            def group_body(g, c2):
                tvec = type_v[j, pl.ds(g * 16, 16)]
                for l in range(16):
                    t = tvec[l]
                    e = g * 16 + l
                    for k in range(D // 16):
                        sl = pl.ds(k * 16, 16)
                        buf[e, sl] = buf[e, sl] * w_v[t, sl]
                return c2
            lax.fori_loop(0, CHUNK // 16, group_body, 0)

        def gather_start(b, j):
            pltpu.async_copy(emb_h.at[tail_v.at[j]], rows[b], gsem[b])

        def gather_wait(b, j):
            pltpu.make_async_copy(emb_h.at[tail_v.at[j]], rows[b], gsem[b]).wait()

        def scatter_start(b, j):
            pltpu.async_copy(rows[b], acc_s.at[head_v.at[j]], ssem[b], add=True)
            pltpu.async_copy(ones_v, cnt_s.at[head_v.at[j]], csem, add=True)

        def scatter_wait(b, j):
            pltpu.make_async_copy(rows[b], acc_s.at[head_v.at[j]], ssem[b]).wait()

        def seg_body(seg, carry0):
            base_c = pl.multiple_of(wid * cpt + seg * SEG, 8)
            pltpu.sync_copy(tail_h.at[pl.ds(base_c, SEG)], tail_v)
            pltpu.sync_copy(head_h.at[pl.ds(base_c, SEG)], head_v)
            pltpu.sync_copy(type_h.at[pl.ds(base_c, SEG)], type_v)

            for b in range(NBUF):
                gather_start(b, b)

            def body(i, carry):
                j0 = i * NBUF
                for b in range(NBUF):
                    gather_wait(b, j0 + b)
                    compute(rows[b], j0 + b)
                    scatter_start(b, j0 + b)
                for b in range(NBUF):
                    scatter_wait(b, j0 + b)
                    gather_start(b, j0 + b + NBUF)
                return carry
            lax.fori_loop(0, steps, body, 0)

            jlast = steps * NBUF
            for b in range(NBUF):
                gather_wait(b, jlast + b)
                compute(rows[b], jlast + b)
                scatter_start(b, jlast + b)
            for b in range(NBUF):
                scatter_wait(b, jlast + b)

            def cdrain(jj, carry):
                pltpu.make_async_copy(ones_v, cnt_s.at[head_v.at[jj]], csem).wait()
                return carry
            lax.fori_loop(0, SEG, cdrain, 0)
            return carry0
        lax.fori_loop(0, NSEG, seg_body, 0)

        plsc.subcore_barrier()

        pltpu.sync_copy(acc_s.at[pl.ds(base_row, ROWS_PER_TILE)],
                        psum_h.at[cid, pl.ds(base_row, ROWS_PER_TILE)])
        pltpu.sync_copy(cnt_s.at[pl.ds(base_row, ROWS_PER_TILE)],
                        pcnt_h.at[cid, pl.ds(base_row, ROWS_PER_TILE)])

    return agg(entity_emb, head2, tail2, type2, weight)


def _combine_kernel(p_ref, c_ref, o_ref):
    s = p_ref[0] + p_ref[1]
    c = c_ref[0] + c_ref[1]
    c = jnp.maximum(c, 1.0)
    o_ref[...] = s / c


def _tc_combine(psum, pcnt):
    blocks = N_PAD // D
    pcnt2 = pcnt.reshape(NC, N_PAD, 1)
    out = pl.pallas_call(
        _combine_kernel,
        grid=(blocks,),
        in_specs=[
            pl.BlockSpec((NC, D, D), lambda i: (0, i, 0)),
            pl.BlockSpec((NC, D, 1), lambda i: (0, i, 0)),
        ],
        out_specs=pl.BlockSpec((D, D), lambda i: (i, 0)),
        out_shape=jax.ShapeDtypeStruct((N_PAD, D), jnp.float32),
    )(psum, pcnt2)
    return out


def kernel(entity_emb, edge_index, edge_type, weight):
    n_entities = entity_emb.shape[0]
    head = edge_index[0].astype(jnp.int32)
    tail = edge_index[1].astype(jnp.int32)
    etype = edge_type.astype(jnp.int32)

    n_edges = head.shape[0]
    span = NW * CHUNK * SEG * NSEG
    n_pad_edges = (-n_edges) % span
    if n_pad_edges:
        head = jnp.concatenate(
            [head, jnp.full((n_pad_edges,), PAD_ROW, jnp.int32)])
        tail = jnp.concatenate([tail, jnp.zeros((n_pad_edges,), jnp.int32)])
        etype = jnp.concatenate([etype, jnp.zeros((n_pad_edges,), jnp.int32)])
    n_chunks = (n_edges + n_pad_edges) // CHUNK

    head2 = head.reshape(n_chunks, CHUNK)
    tail2 = tail.reshape(n_chunks, CHUNK)
    type2 = etype.reshape(n_chunks, CHUNK)

    psum, pcnt = _sc_partials(entity_emb, head2, tail2, type2, weight)
    out = _tc_combine(psum, pcnt)
    return out[:n_entities]


# speedup vs baseline: 1.1150x; 1.1150x over previous
import functools

import jax
import jax.numpy as jnp
from jax import lax
from jax.experimental import pallas as pl
from jax.experimental.pallas import tpu as pltpu
from jax.experimental.pallas import tpu_sc as plsc

D = 128
DH = D // 2
NREL = 32
NC, NS = 2, 16
CHUNK = 128
NBUF = 4
SEG = 32
NSEG = 5
ROWS_PER_TILE = 640
N_PAD = NS * ROWS_PER_TILE
PAD_ROW = 10100


def _sc_partials(emb2, head2, tail2, type2, wsplit):
    cpt = SEG * NSEG
    steps = SEG // NBUF - 1

    mesh = plsc.VectorSubcoreMesh(core_axis_name="c", subcore_axis_name="s")

    @functools.partial(
        pl.kernel,
        out_type=(
            jax.ShapeDtypeStruct((NC, N_PAD, DH), jnp.float32),
            jax.ShapeDtypeStruct((NC, N_PAD), jnp.float32),
        ),
        mesh=mesh,
        compiler_params=pltpu.CompilerParams(use_tc_tiling_on_sc=False),
        scratch_types=[
            pltpu.VMEM_SHARED((N_PAD, DH), jnp.float32),
            pltpu.VMEM_SHARED((N_PAD,), jnp.float32),
            pltpu.VMEM((NREL, DH), jnp.float32),
            pltpu.VMEM((SEG, CHUNK), jnp.int32),
            pltpu.VMEM((SEG, CHUNK), jnp.int32),
            pltpu.VMEM((SEG, CHUNK), jnp.int32),
            pltpu.VMEM((CHUNK, DH), jnp.float32),
            pltpu.VMEM((CHUNK, DH), jnp.float32),
            pltpu.VMEM((CHUNK, DH), jnp.float32),
            pltpu.VMEM((CHUNK, DH), jnp.float32),
            pltpu.VMEM((CHUNK,), jnp.float32),
            pltpu.VMEM((ROWS_PER_TILE,), jnp.float32),
            pltpu.SemaphoreType.DMA,
            pltpu.SemaphoreType.DMA,
            pltpu.SemaphoreType.DMA,
            pltpu.SemaphoreType.DMA,
            pltpu.SemaphoreType.DMA,
            pltpu.SemaphoreType.DMA,
            pltpu.SemaphoreType.DMA,
            pltpu.SemaphoreType.DMA,
            pltpu.SemaphoreType.DMA,
        ],
    )
    def agg(emb_h, head_h, tail_h, type_h, w_h, psum_h, pcnt_h,
            acc_s, cnt_s, w_v, tail_v, head_v, type_v,
            rows0, rows1, rows2, rows3, ones_v, zcnt_v,
            g0, g1, g2, g3, s0, s1, s2, s3, csem):
        cid = lax.axis_index("c")
        sid = lax.axis_index("s")
        rows = [rows0, rows1, rows2, rows3]
        gsem = [g0, g1, g2, g3]
        ssem = [s0, s1, s2, s3]

        def zrow(i, carry):
            for k in range(DH // 16):
                rows0[i, pl.ds(k * 16, 16)] = jnp.zeros((16,), jnp.float32)
            return carry
        lax.fori_loop(0, CHUNK, zrow, 0)
        for i in range(CHUNK // 16):
            ones_v[pl.ds(i * 16, 16)] = jnp.ones((16,), jnp.float32)
        for i in range(ROWS_PER_TILE // 16):
            zcnt_v[pl.ds(i * 16, 16)] = jnp.zeros((16,), jnp.float32)

        pltpu.sync_copy(w_h.at[cid], w_v)

        base_row = sid * ROWS_PER_TILE
        for b in range(ROWS_PER_TILE // CHUNK):
            pltpu.sync_copy(rows0, acc_s.at[pl.ds(base_row + b * CHUNK, CHUNK)])
        pltpu.sync_copy(zcnt_v, cnt_s.at[pl.ds(base_row, ROWS_PER_TILE)])
        plsc.subcore_barrier()

        def compute(buf, j):
            def group_body(g, c2):
                tvec = type_v[j, pl.ds(g * 16, 16)]
                for l in range(16):
                    t = tvec[l]
                    e = g * 16 + l
                    for k in range(DH // 16):
                        sl = pl.ds(k * 16, 16)
                        buf[e, sl] = buf[e, sl] * w_v[t, sl]
                return c2
            lax.fori_loop(0, CHUNK // 16, group_body, 0)

        def gather_start(b, j):
            pltpu.async_copy(emb_h.at[tail_v.at[j]], rows[b], gsem[b])

        def gather_wait(b, j):
            pltpu.make_async_copy(emb_h.at[tail_v.at[j]], rows[b], gsem[b]).wait()

        def scatter_start(b, j):
            pltpu.async_copy(rows[b], acc_s.at[head_v.at[j]], ssem[b], add=True)
            pltpu.async_copy(ones_v, cnt_s.at[head_v.at[j]], csem, add=True)

        def scatter_wait(b, j):
            pltpu.make_async_copy(rows[b], acc_s.at[head_v.at[j]], ssem[b]).wait()

        def seg_body(seg, carry0):
            base_c = pl.multiple_of(sid * cpt + seg * SEG, 8)
            pltpu.sync_copy(tail_h.at[pl.ds(base_c, SEG)], tail_v)
            pltpu.sync_copy(head_h.at[pl.ds(base_c, SEG)], head_v)
            pltpu.sync_copy(type_h.at[pl.ds(base_c, SEG)], type_v)

            def tmap(i, carry):
                r = i >> 3
                c = (i & 7) * 16
                v = tail_v[r, pl.ds(c, 16)]
                tail_v[r, pl.ds(c, 16)] = v * 2 + cid
                return carry
            lax.fori_loop(0, SEG * (CHUNK // 16), tmap, 0)

            for b in range(NBUF):
                gather_start(b, b)

            def body(i, carry):
                j0 = i * NBUF
                for b in range(NBUF):
                    gather_wait(b, j0 + b)
                    compute(rows[b], j0 + b)
                    scatter_start(b, j0 + b)
                for b in range(NBUF):
                    scatter_wait(b, j0 + b)
                    gather_start(b, j0 + b + NBUF)
                return carry
            lax.fori_loop(0, steps, body, 0)

            jlast = steps * NBUF
            for b in range(NBUF):
                gather_wait(b, jlast + b)
                compute(rows[b], jlast + b)
                scatter_start(b, jlast + b)
            for b in range(NBUF):
                scatter_wait(b, jlast + b)

            def cdrain(jj, carry):
                pltpu.make_async_copy(ones_v, cnt_s.at[head_v.at[jj]], csem).wait()
                return carry
            lax.fori_loop(0, SEG, cdrain, 0)
            return carry0
        lax.fori_loop(0, NSEG, seg_body, 0)

        plsc.subcore_barrier()

        pltpu.sync_copy(acc_s.at[pl.ds(base_row, ROWS_PER_TILE)],
                        psum_h.at[cid, pl.ds(base_row, ROWS_PER_TILE)])
        pltpu.sync_copy(cnt_s.at[pl.ds(base_row, ROWS_PER_TILE)],
                        pcnt_h.at[cid, pl.ds(base_row, ROWS_PER_TILE)])

    return agg(emb2, head2, tail2, type2, wsplit)


def _combine_kernel(p_ref, c_ref, o_ref):
    c = jnp.maximum(c_ref[0], 1.0)
    o_ref[:, :DH] = p_ref[0] / c
    o_ref[:, DH:] = p_ref[1] / c


def _tc_combine(psum, pcnt):
    blocks = N_PAD // D
    pcnt2 = pcnt.reshape(NC, N_PAD, 1)
    out = pl.pallas_call(
        _combine_kernel,
        grid=(blocks,),
        in_specs=[
            pl.BlockSpec((NC, D, DH), lambda i: (0, i, 0)),
            pl.BlockSpec((1, D, 1), lambda i: (0, i, 0)),
        ],
        out_specs=pl.BlockSpec((D, D), lambda i: (i, 0)),
        out_shape=jax.ShapeDtypeStruct((N_PAD, D), jnp.float32),
    )(psum, pcnt2)
    return out


def kernel(entity_emb, edge_index, edge_type, weight):
    n_entities = entity_emb.shape[0]
    head = edge_index[0].astype(jnp.int32)
    tail = edge_index[1].astype(jnp.int32)
    etype = edge_type.astype(jnp.int32)

    n_edges = head.shape[0]
    span = NS * CHUNK * SEG * NSEG
    n_pad_edges = (-n_edges) % span
    if n_pad_edges:
        head = jnp.concatenate(
            [head, jnp.full((n_pad_edges,), PAD_ROW, jnp.int32)])
        tail = jnp.concatenate([tail, jnp.zeros((n_pad_edges,), jnp.int32)])
        etype = jnp.concatenate([etype, jnp.zeros((n_pad_edges,), jnp.int32)])
    n_chunks = (n_edges + n_pad_edges) // CHUNK

    head2 = head.reshape(n_chunks, CHUNK)
    tail2 = tail.reshape(n_chunks, CHUNK)
    type2 = etype.reshape(n_chunks, CHUNK)

    emb2 = entity_emb.reshape(2 * n_entities, DH)
    wsplit = weight.reshape(NREL, NC, DH).transpose(1, 0, 2)

    psum, pcnt = _sc_partials(emb2, head2, tail2, type2, wsplit)
    out = _tc_combine(psum, pcnt)
    return out[:n_entities]


# speedup vs baseline: 1.1569x; 1.0375x over previous
import functools

import jax
import jax.numpy as jnp
from jax import lax
from jax.experimental import pallas as pl
from jax.experimental.pallas import tpu as pltpu
from jax.experimental.pallas import tpu_sc as plsc

D = 128
DH = D // 2
NREL = 32
NC, NS = 2, 16
CHUNK = 128
NBUF = 8
SEG = 32
NSEG = 5
ROWS_PER_TILE = 640
N_PAD = NS * ROWS_PER_TILE
PAD_ROW = 10100


def _sc_partials(emb2, head2, tail2, type2, wsplit):
    cpt = SEG * NSEG
    steps = SEG // NBUF - 1

    mesh = plsc.VectorSubcoreMesh(core_axis_name="c", subcore_axis_name="s")

    @functools.partial(
        pl.kernel,
        out_type=(
            jax.ShapeDtypeStruct((NC, N_PAD, DH), jnp.float32),
            jax.ShapeDtypeStruct((NC, N_PAD), jnp.float32),
        ),
        mesh=mesh,
        compiler_params=pltpu.CompilerParams(use_tc_tiling_on_sc=False),
        scratch_types=[
            pltpu.VMEM_SHARED((N_PAD, DH), jnp.float32),
            pltpu.VMEM_SHARED((N_PAD,), jnp.float32),
            pltpu.VMEM((NREL, DH), jnp.float32),
            pltpu.VMEM((SEG, CHUNK), jnp.int32),
            pltpu.VMEM((SEG, CHUNK), jnp.int32),
            pltpu.VMEM((SEG, CHUNK), jnp.int32),
            pltpu.VMEM((CHUNK, DH), jnp.float32),
            pltpu.VMEM((CHUNK, DH), jnp.float32),
            pltpu.VMEM((CHUNK, DH), jnp.float32),
            pltpu.VMEM((CHUNK, DH), jnp.float32),
            pltpu.VMEM((CHUNK, DH), jnp.float32),
            pltpu.VMEM((CHUNK, DH), jnp.float32),
            pltpu.VMEM((CHUNK, DH), jnp.float32),
            pltpu.VMEM((CHUNK, DH), jnp.float32),
            pltpu.VMEM((CHUNK,), jnp.float32),
            pltpu.VMEM((ROWS_PER_TILE,), jnp.float32),
            pltpu.SemaphoreType.DMA,
            pltpu.SemaphoreType.DMA,
            pltpu.SemaphoreType.DMA,
            pltpu.SemaphoreType.DMA,
            pltpu.SemaphoreType.DMA,
            pltpu.SemaphoreType.DMA,
            pltpu.SemaphoreType.DMA,
            pltpu.SemaphoreType.DMA,
            pltpu.SemaphoreType.DMA,
            pltpu.SemaphoreType.DMA,
            pltpu.SemaphoreType.DMA,
            pltpu.SemaphoreType.DMA,
            pltpu.SemaphoreType.DMA,
            pltpu.SemaphoreType.DMA,
            pltpu.SemaphoreType.DMA,
            pltpu.SemaphoreType.DMA,
            pltpu.SemaphoreType.DMA,
        ],
    )
    def agg(emb_h, head_h, tail_h, type_h, w_h, psum_h, pcnt_h,
            acc_s, cnt_s, w_v, tail_v, head_v, type_v,
            rows0, rows1, rows2, rows3, rows4, rows5, rows6, rows7,
            ones_v, zcnt_v,
            g0, g1, g2, g3, g4, g5, g6, g7,
            s0, s1, s2, s3, s4, s5, s6, s7, csem):
        cid = lax.axis_index("c")
        sid = lax.axis_index("s")
        rows = [rows0, rows1, rows2, rows3, rows4, rows5, rows6, rows7]
        gsem = [g0, g1, g2, g3, g4, g5, g6, g7]
        ssem = [s0, s1, s2, s3, s4, s5, s6, s7]

        def zrow(i, carry):
            for k in range(DH // 16):
                rows0[i, pl.ds(k * 16, 16)] = jnp.zeros((16,), jnp.float32)
            return carry
        lax.fori_loop(0, CHUNK, zrow, 0)
        for i in range(CHUNK // 16):
            ones_v[pl.ds(i * 16, 16)] = jnp.ones((16,), jnp.float32)
        for i in range(ROWS_PER_TILE // 16):
            zcnt_v[pl.ds(i * 16, 16)] = jnp.zeros((16,), jnp.float32)

        pltpu.sync_copy(w_h.at[cid], w_v)

        base_row = sid * ROWS_PER_TILE
        for b in range(ROWS_PER_TILE // CHUNK):
            pltpu.sync_copy(rows0, acc_s.at[pl.ds(base_row + b * CHUNK, CHUNK)])
        pltpu.sync_copy(zcnt_v, cnt_s.at[pl.ds(base_row, ROWS_PER_TILE)])
        plsc.subcore_barrier()

        def compute(buf, j):
            def group_body(g, c2):
                tvec = type_v[j, pl.ds(g * 16, 16)]
                for l in range(16):
                    t = tvec[l]
                    e = g * 16 + l
                    for k in range(DH // 16):
                        sl = pl.ds(k * 16, 16)
                        buf[e, sl] = buf[e, sl] * w_v[t, sl]
                return c2
            lax.fori_loop(0, CHUNK // 16, group_body, 0)

        def gather_start(b, j):
            pltpu.async_copy(emb_h.at[tail_v.at[j]], rows[b], gsem[b])

        def gather_wait(b, j):
            pltpu.make_async_copy(emb_h.at[tail_v.at[j]], rows[b], gsem[b]).wait()

        def scatter_start(b, j):
            pltpu.async_copy(rows[b], acc_s.at[head_v.at[j]], ssem[b], add=True)
            pltpu.async_copy(ones_v, cnt_s.at[head_v.at[j]], csem, add=True)

        def scatter_wait(b, j):
            pltpu.make_async_copy(rows[b], acc_s.at[head_v.at[j]], ssem[b]).wait()

        def seg_body(seg, carry0):
            base_c = pl.multiple_of(sid * cpt + seg * SEG, 8)
            pltpu.sync_copy(tail_h.at[pl.ds(base_c, SEG)], tail_v)
            pltpu.sync_copy(head_h.at[pl.ds(base_c, SEG)], head_v)
            pltpu.sync_copy(type_h.at[pl.ds(base_c, SEG)], type_v)

            def tmap(i, carry):
                r = i >> 3
                c = (i & 7) * 16
                v = tail_v[r, pl.ds(c, 16)]
                tail_v[r, pl.ds(c, 16)] = v * 2 + cid
                return carry
            lax.fori_loop(0, SEG * (CHUNK // 16), tmap, 0)

            for b in range(NBUF):
                gather_start(b, b)

            def body(i, carry):
                j0 = i * NBUF
                for b in range(NBUF):
                    gather_wait(b, j0 + b)
                    compute(rows[b], j0 + b)
                    scatter_start(b, j0 + b)
                for b in range(NBUF):
                    scatter_wait(b, j0 + b)
                    gather_start(b, j0 + b + NBUF)
                return carry
            lax.fori_loop(0, steps, body, 0)

            jlast = steps * NBUF
            for b in range(NBUF):
                gather_wait(b, jlast + b)
                compute(rows[b], jlast + b)
                scatter_start(b, jlast + b)
            for b in range(NBUF):
                scatter_wait(b, jlast + b)

            def cdrain(jj, carry):
                pltpu.make_async_copy(ones_v, cnt_s.at[head_v.at[jj]], csem).wait()
                return carry
            lax.fori_loop(0, SEG, cdrain, 0)
            return carry0
        lax.fori_loop(0, NSEG, seg_body, 0)

        plsc.subcore_barrier()

        pltpu.sync_copy(acc_s.at[pl.ds(base_row, ROWS_PER_TILE)],
                        psum_h.at[cid, pl.ds(base_row, ROWS_PER_TILE)])
        pltpu.sync_copy(cnt_s.at[pl.ds(base_row, ROWS_PER_TILE)],
                        pcnt_h.at[cid, pl.ds(base_row, ROWS_PER_TILE)])

    return agg(emb2, head2, tail2, type2, wsplit)


def _combine_kernel(p_ref, c_ref, o_ref):
    c = jnp.maximum(c_ref[0], 1.0)
    o_ref[:, :DH] = p_ref[0] / c
    o_ref[:, DH:] = p_ref[1] / c


def _tc_combine(psum, pcnt):
    blocks = N_PAD // D
    pcnt2 = pcnt.reshape(NC, N_PAD, 1)
    out = pl.pallas_call(
        _combine_kernel,
        grid=(blocks,),
        in_specs=[
            pl.BlockSpec((NC, D, DH), lambda i: (0, i, 0)),
            pl.BlockSpec((1, D, 1), lambda i: (0, i, 0)),
        ],
        out_specs=pl.BlockSpec((D, D), lambda i: (i, 0)),
        out_shape=jax.ShapeDtypeStruct((N_PAD, D), jnp.float32),
    )(psum, pcnt2)
    return out


def kernel(entity_emb, edge_index, edge_type, weight):
    n_entities = entity_emb.shape[0]
    head = edge_index[0].astype(jnp.int32)
    tail = edge_index[1].astype(jnp.int32)
    etype = edge_type.astype(jnp.int32)

    n_edges = head.shape[0]
    span = NS * CHUNK * SEG * NSEG
    n_pad_edges = (-n_edges) % span
    if n_pad_edges:
        head = jnp.concatenate(
            [head, jnp.full((n_pad_edges,), PAD_ROW, jnp.int32)])
        tail = jnp.concatenate([tail, jnp.zeros((n_pad_edges,), jnp.int32)])
        etype = jnp.concatenate([etype, jnp.zeros((n_pad_edges,), jnp.int32)])
    n_chunks = (n_edges + n_pad_edges) // CHUNK

    head2 = head.reshape(n_chunks, CHUNK)
    tail2 = tail.reshape(n_chunks, CHUNK)
    type2 = etype.reshape(n_chunks, CHUNK)

    emb2 = entity_emb.reshape(2 * n_entities, DH)
    wsplit = weight.reshape(NREL, NC, DH).transpose(1, 0, 2)

    psum, pcnt = _sc_partials(emb2, head2, tail2, type2, wsplit)
    out = _tc_combine(psum, pcnt)
    return out[:n_entities]


# speedup vs baseline: 1.1896x; 1.0283x over previous
import functools

import jax
import jax.numpy as jnp
from jax import lax
from jax.experimental import pallas as pl
from jax.experimental.pallas import tpu as pltpu
from jax.experimental.pallas import tpu_sc as plsc

D = 128
NREL = 32
NC, NS = 2, 16
NW = NC * NS
CHUNK = 128
ROWS_PER_TILE = 640
N_PAD = NS * ROWS_PER_TILE


def _sc_partials(entity_emb, head, tail, etype, weight):
    n_edges = head.shape[0]
    n_chunks = n_edges // CHUNK

    mesh = plsc.VectorSubcoreMesh(core_axis_name="c", subcore_axis_name="s")

    @functools.partial(
        pl.kernel,
        out_type=(
            jax.ShapeDtypeStruct((NC, N_PAD, D), jnp.float32),
            jax.ShapeDtypeStruct((NC, N_PAD), jnp.float32),
        ),
        mesh=mesh,
        scratch_types=[
            pltpu.VMEM_SHARED((N_PAD, D), jnp.float32),
            pltpu.VMEM_SHARED((N_PAD,), jnp.float32),
            pltpu.VMEM((NREL, D), jnp.float32),
            pltpu.VMEM((CHUNK,), jnp.int32),
            pltpu.VMEM((CHUNK,), jnp.int32),
            pltpu.VMEM((CHUNK,), jnp.int32),
            pltpu.VMEM((CHUNK, D), jnp.float32),
            pltpu.VMEM((CHUNK,), jnp.float32),
            pltpu.VMEM((ROWS_PER_TILE,), jnp.float32),
            pltpu.SemaphoreType.DMA,
        ],
    )
    def agg(emb_h, head_h, tail_h, type_h, w_h, psum_h, pcnt_h,
            acc_s, cnt_s, w_v, tail_v, head_v, type_v, rows_v, ones_v,
            zcnt_v, sem):
        cid = lax.axis_index("c")
        sid = lax.axis_index("s")
        wid = sid * NC + cid

        def zrow(i, carry):
            for k in range(D // 16):
                rows_v[i, pl.ds(k * 16, 16)] = jnp.zeros((16,), jnp.float32)
            return carry
        lax.fori_loop(0, CHUNK, zrow, 0)

        def zsmall(i, carry):
            ones_v[pl.ds(i * 16, 16)] = jnp.ones((16,), jnp.float32)
            return carry
        lax.fori_loop(0, CHUNK // 16, zsmall, 0)

        def zcnt(i, carry):
            zcnt_v[pl.ds(i * 16, 16)] = jnp.zeros((16,), jnp.float32)
            return carry
        lax.fori_loop(0, ROWS_PER_TILE // 16, zcnt, 0)

        pltpu.sync_copy(w_h, w_v)

        base_row = sid * ROWS_PER_TILE
        for b in range(ROWS_PER_TILE // CHUNK):
            pltpu.sync_copy(rows_v, acc_s.at[pl.ds(base_row + b * CHUNK, CHUNK)])
        pltpu.sync_copy(zcnt_v, cnt_s.at[pl.ds(base_row, ROWS_PER_TILE)])
        plsc.subcore_barrier()

        my_chunks = n_chunks // NW + jnp.where(wid < (n_chunks % NW), 1, 0)

        def chunk_body(j, carry):
            chunk_id = wid + j * NW
            base = chunk_id * CHUNK
            pltpu.sync_copy(tail_h.at[pl.ds(base, CHUNK)], tail_v)
            pltpu.sync_copy(head_h.at[pl.ds(base, CHUNK)], head_v)
            pltpu.sync_copy(type_h.at[pl.ds(base, CHUNK)], type_v)
            pltpu.async_copy(emb_h.at[tail_v], rows_v, sem).wait()

            def group_body(g, c2):
                tvec = type_v[pl.ds(g * 16, 16)]
                for l in range(16):
                    t = tvec[l]
                    e = g * 16 + l
                    for k in range(D // 16):
                        sl = pl.ds(k * 16, 16)
                        rows_v[e, sl] = rows_v[e, sl] * w_v[t, sl]
                return c2
            lax.fori_loop(0, CHUNK // 16, group_body, 0)

            pltpu.sync_copy(rows_v, acc_s.at[head_v], add=True)
            pltpu.sync_copy(ones_v, cnt_s.at[head_v], add=True)
            return carry
        lax.fori_loop(0, my_chunks, chunk_body, 0)

        plsc.subcore_barrier()

        pltpu.sync_copy(acc_s.at[pl.ds(base_row, ROWS_PER_TILE)],
                        psum_h.at[cid, pl.ds(base_row, ROWS_PER_TILE)])
        pltpu.sync_copy(cnt_s.at[pl.ds(base_row, ROWS_PER_TILE)],
                        pcnt_h.at[cid, pl.ds(base_row, ROWS_PER_TILE)])

    return agg(entity_emb, head, tail, etype, weight)


def _combine_kernel(p_ref, c_ref, o_ref):
    s = p_ref[0] + p_ref[1]
    c = c_ref[0] + c_ref[1]
    c = jnp.maximum(c, 1.0)
    o_ref[...] = s / c


def _tc_combine(psum, pcnt):
    blocks = N_PAD // D
    pcnt2 = pcnt.reshape(NC, N_PAD, 1)
    out = pl.pallas_call(
        _combine_kernel,
        grid=(blocks,),
        in_specs=[
            pl.BlockSpec((NC, D, D), lambda i: (0, i, 0)),
            pl.BlockSpec((NC, D, 1), lambda i: (0, i, 0)),
        ],
        out_specs=pl.BlockSpec((D, D), lambda i: (i, 0)),
        out_shape=jax.ShapeDtypeStruct((N_PAD, D), jnp.float32),
    )(psum, pcnt2)
    return out


def kernel(entity_emb, edge_index, edge_type, weight):
    n_entities = entity_emb.shape[0]
    head = edge_index[0].astype(jnp.int32)
    tail = edge_index[1].astype(jnp.int32)
    etype = edge_type.astype(jnp.int32)
    psum, pcnt = _sc_partials(entity_emb, head, tail, etype, weight)
    out = _tc_combine(psum, pcnt)
    return out[:n_entities]
